# baseline (speedup 1.0000x reference)
import functools

import jax
import jax.numpy as jnp
from jax import lax
from jax.experimental import pallas as pl
from jax.experimental.pallas import tpu as pltpu
from jax.experimental.pallas import tpu_sc as plsc

EMBEDDING_DIM = 64
SCALE = 8.0
NBUF = 8


def kernel(x, table):
    R, C = x.shape
    xi = x.astype(jnp.int32)

    info = plsc.get_sparse_core_info()
    NC, NS = info.num_cores, info.num_subcores
    NW = NC * NS
    assert R % NW == 0
    rows_per_w = R // NW
    assert rows_per_w % NBUF == 0 and rows_per_w >= 2 * NBUF

    mesh = plsc.VectorSubcoreMesh(core_axis_name="c", subcore_axis_name="s")

    @functools.partial(
        pl.kernel,
        mesh=mesh,
        compiler_params=pltpu.CompilerParams(use_tc_tiling_on_sc=False),
        out_type=jax.ShapeDtypeStruct((R, C, EMBEDDING_DIM), jnp.float32),
        scratch_types=[
            pltpu.VMEM((rows_per_w, C), jnp.int32),
            pltpu.VMEM((NBUF, C, EMBEDDING_DIM), jnp.float32),
            pltpu.VMEM((NBUF, C, EMBEDDING_DIM), jnp.float32),
        ]
        + [pltpu.SemaphoreType.DMA] * (2 * NBUF),
    )
    def emb_kernel(idx_hbm, table_hbm, out_hbm, idx_v, gb, sb, *sems):
        gsem = sems[:NBUF]
        ssem = sems[NBUF:]
        wid = lax.axis_index("s") * NC + lax.axis_index("c")
        base = wid * rows_per_w
        pltpu.sync_copy(idx_hbm.at[pl.ds(base, rows_per_w), :], idx_v)

        def start_gather(j, b):
            pltpu.async_copy(table_hbm.at[idx_v.at[j]], gb.at[b], gsem[b])

        def wait_gather(b):
            pltpu.make_async_copy(
                table_hbm.at[idx_v.at[0]], gb.at[b], gsem[b]).wait()

        def start_scatter(j, b):
            pltpu.async_copy(sb.at[b], out_hbm.at[base + j], ssem[b])

        def wait_scatter(b):
            pltpu.make_async_copy(sb.at[b], out_hbm.at[base], ssem[b]).wait()

        def scale(b):
            def row_body(r, carry):
                for rr in range(2):
                    for c in range(EMBEDDING_DIM // 16):
                        sl = pl.ds(c * 16, 16)
                        sb[b, r + rr, sl] = gb[b, r + rr, sl] * SCALE
                return carry

            lax.fori_loop(0, C // 2, lambda r, c2: row_body(2 * r, c2), 0,
                          unroll=2)

        for b in range(NBUF):
            start_gather(b, b)
        for b in range(NBUF):
            wait_gather(b)
            scale(b)
            start_scatter(b, b)
            start_gather(NBUF + b, b)

        def main_body(j0, carry):
            for b in range(NBUF):
                j = j0 + b
                wait_gather(b)
                wait_scatter(b)
                scale(b)
                start_scatter(j, b)
                start_gather(j + NBUF, b)
            return carry

        lax.fori_loop(1, rows_per_w // NBUF - 1,
                      lambda t, c2: main_body(t * NBUF, c2), 0)

        j0 = rows_per_w - NBUF
        for b in range(NBUF):
            wait_gather(b)
            wait_scatter(b)
            scale(b)
            start_scatter(j0 + b, b)
        for b in range(NBUF):
            wait_scatter(b)

    return emb_kernel(xi, table)


# speedup vs baseline: 1.1210x; 1.1210x over previous
import functools

import jax
import jax.numpy as jnp
from jax import lax
from jax.experimental import pallas as pl
from jax.experimental.pallas import tpu as pltpu
from jax.experimental.pallas import tpu_sc as plsc

EMBEDDING_DIM = 64
SCALE = 8.0
CHUNK = 128
NBUF = 4


def kernel(x, table):
    B = x.shape[0] * x.shape[1]
    N = table.shape[0]
    xf = x.reshape(-1).astype(jnp.int32)
    tp = jnp.pad(table, ((0, 0), (0, 128 - EMBEDDING_DIM)))
    tp = tp.reshape(2 * N, EMBEDDING_DIM)

    info = plsc.get_sparse_core_info()
    NC, NS = info.num_cores, info.num_subcores
    NW = NC * NS
    assert B % (NW * CHUNK) == 0
    b_per_w = B // NW
    n_chunks = b_per_w // CHUNK
    assert n_chunks % NBUF == 0 and n_chunks >= 2 * NBUF

    mesh = plsc.VectorSubcoreMesh(core_axis_name="c", subcore_axis_name="s")

    @functools.partial(
        pl.kernel,
        mesh=mesh,
        compiler_params=pltpu.CompilerParams(use_tc_tiling_on_sc=False),
        out_type=jax.ShapeDtypeStruct((B, EMBEDDING_DIM), jnp.float32),
        scratch_types=[
            pltpu.VMEM((b_per_w,), jnp.int32),
            pltpu.VMEM((NBUF, CHUNK, EMBEDDING_DIM), jnp.float32),
            pltpu.VMEM((NBUF, CHUNK, EMBEDDING_DIM), jnp.float32),
        ]
        + [pltpu.SemaphoreType.DMA] * (2 * NBUF),
    )
    def emb_kernel(idx_hbm, table_hbm, out_hbm, idx_v, gb, sb, *sems):
        gsem = sems[:NBUF]
        ssem = sems[NBUF:]
        wid = lax.axis_index("s") * NC + lax.axis_index("c")
        base = wid * b_per_w
        pltpu.sync_copy(idx_hbm.at[pl.ds(base, b_per_w)], idx_v)

        def dbl(k, carry):
            sl = pl.ds(k * 16, 16)
            idx_v[sl] = idx_v[sl] * 2
            return carry

        lax.fori_loop(0, b_per_w // 16, dbl, 0, unroll=4)

        def start_gather(j, b):
            pltpu.async_copy(
                table_hbm.at[idx_v.at[pl.ds(j * CHUNK, CHUNK)]],
                gb.at[b], gsem[b])

        def wait_gather(b):
            pltpu.make_async_copy(
                table_hbm.at[idx_v.at[pl.ds(0, CHUNK)]], gb.at[b],
                gsem[b]).wait()

        def start_scatter(j, b):
            pltpu.async_copy(
                sb.at[b], out_hbm.at[pl.ds(base + j * CHUNK, CHUNK)], ssem[b])

        def wait_scatter(b):
            pltpu.make_async_copy(
                sb.at[b], out_hbm.at[pl.ds(0, CHUNK)], ssem[b]).wait()

        def scale(b):
            def row_body(r, carry):
                for rr in range(2):
                    for c in range(EMBEDDING_DIM // 16):
                        sl = pl.ds(c * 16, 16)
                        sb[b, r + rr, sl] = gb[b, r + rr, sl] * SCALE
                return carry

            lax.fori_loop(0, CHUNK // 2, lambda r, c2: row_body(2 * r, c2), 0,
                          unroll=2)

        for b in range(NBUF):
            start_gather(b, b)
        for b in range(NBUF):
            wait_gather(b)
            scale(b)
            start_scatter(b, b)
            start_gather(NBUF + b, b)

        def main_body(j0, carry):
            for b in range(NBUF):
                j = j0 + b
                wait_gather(b)
                wait_scatter(b)
                scale(b)
                start_scatter(j, b)
                start_gather(j + NBUF, b)
            return carry

        lax.fori_loop(1, n_chunks // NBUF - 1,
                      lambda t, c2: main_body(t * NBUF, c2), 0)

        j0 = n_chunks - NBUF
        for b in range(NBUF):
            wait_gather(b)
            wait_scatter(b)
            scale(b)
            start_scatter(j0 + b, b)
        for b in range(NBUF):
            wait_scatter(b)

    out = emb_kernel(xf, tp)
    return out.reshape(x.shape[0], x.shape[1], EMBEDDING_DIM)
